# Initial kernel scaffold; baseline (speedup 1.0000x reference)
import jax
import jax.numpy as jnp
from jax.experimental import pallas as pl


def kernel(b_coords, b_o, b_scores):
    raise NotImplementedError("write your pallas kernel here")


# speedup vs baseline: 21.7562x; 21.7562x over previous
import jax
import jax.numpy as jnp
from jax import lax
from jax.experimental import pallas as pl

_NMS_TH = 0.7
_SCORE_TH = 0.05
_N = 5000
_R = 8
_C = 625


def _nms_body(coords_ref, o_ref, sc_ref, res_ref, lab_ref):
    sc = sc_ref[0]
    cls = jnp.max(sc, axis=0)
    cio = lax.broadcasted_iota(jnp.int32, sc.shape, 0)
    lbl = jnp.min(jnp.where(sc == cls, cio, sc.shape[0]), axis=0)

    obj = o_ref[0, 0]
    score = cls * obj
    selm = (obj >= 0.5) & (score >= _SCORE_TH)

    xc = coords_ref[0, 0]
    yc = coords_ref[0, 1]
    w = coords_ref[0, 2]
    h = coords_ref[0, 3]
    x0p = jnp.clip(xc - w * 0.5, 0.0, 1.0)
    y0p = jnp.clip(yc - h * 0.5, 0.0, 1.0)
    x1p = jnp.clip(xc + w * 0.5, 0.0, 1.0)
    y1p = jnp.clip(yc + h * 0.5, 0.0, 1.0)
    ap = jnp.maximum(x1p - x0p, 0.0) * jnp.maximum(y1p - y0p, 0.0)

    idxp = (lax.broadcasted_iota(jnp.int32, (_R, _C), 0) * _C
            + lax.broadcasted_iota(jnp.int32, (_R, _C), 1))
    ms0 = jnp.where(selm, score, -1.0)
    keep0 = jnp.zeros((_R, _C), jnp.float32)

    def cond(c):
        ms, _ = c
        return jnp.max(ms) >= 0.0

    def body(c):
        ms, keep = c
        m = jnp.max(ms)
        pidx = jnp.min(jnp.where(ms == m, idxp, _N))
        onehot = idxp == pidx
        neg = jnp.float32(-1.0)
        px0 = jnp.max(jnp.where(onehot, x0p, neg))
        py0 = jnp.max(jnp.where(onehot, y0p, neg))
        px1 = jnp.max(jnp.where(onehot, x1p, neg))
        py1 = jnp.max(jnp.where(onehot, y1p, neg))
        pa = jnp.max(jnp.where(onehot, ap, neg))
        iw = jnp.maximum(jnp.minimum(px1, x1p) - jnp.maximum(px0, x0p), 0.0)
        ih = jnp.maximum(jnp.minimum(py1, y1p) - jnp.maximum(py0, y0p), 0.0)
        inter = iw * ih
        supp = inter * (1.0 + _NMS_TH) > _NMS_TH * (pa + ap)
        keep = jnp.where(onehot, 1.0, keep)
        ms = jnp.where(supp | onehot, -1.0, ms)
        return ms, keep

    _, kf = lax.while_loop(cond, body, (ms0, keep0))

    keep = kf > 0.0
    res_ref[0, 0] = x0p * kf
    res_ref[0, 1] = y0p * kf
    res_ref[0, 2] = x1p * kf
    res_ref[0, 3] = y1p * kf
    res_ref[0, 4] = score * kf
    lab_ref[0, 0] = jnp.where(keep, lbl, 0)


def kernel(b_coords, b_o, b_scores):
    B, N, _ = b_coords.shape
    coords_t = b_coords.transpose(0, 2, 1).reshape(B, 4, _R, _C)
    o4 = b_o.reshape(B, 1, _R, _C)
    scores_t = b_scores.transpose(0, 2, 1).reshape(B, 80, _R, _C)

    res, lab = pl.pallas_call(
        _nms_body,
        grid=(B,),
        in_specs=[
            pl.BlockSpec((1, 4, _R, _C), lambda b: (b, 0, 0, 0)),
            pl.BlockSpec((1, 1, _R, _C), lambda b: (b, 0, 0, 0)),
            pl.BlockSpec((1, 80, _R, _C), lambda b: (b, 0, 0, 0)),
        ],
        out_specs=[
            pl.BlockSpec((1, 5, _R, _C), lambda b: (b, 0, 0, 0)),
            pl.BlockSpec((1, 1, _R, _C), lambda b: (b, 0, 0, 0)),
        ],
        out_shape=[
            jax.ShapeDtypeStruct((B, 5, _R, _C), jnp.float32),
            jax.ShapeDtypeStruct((B, 1, _R, _C), jnp.int32),
        ],
    )(coords_t, o4, scores_t)

    out = res.reshape(B, 5, N).transpose(0, 2, 1)
    labels = lab.reshape(B, N)
    return out, labels


# speedup vs baseline: 87.4394x; 4.0190x over previous
import jax
import jax.numpy as jnp
from jax import lax
from jax.experimental import pallas as pl

_NMS_TH = 0.7
_SCORE_TH = 0.05
_N = 5000
_R = 8
_C = 625


def _nms_body(coords_ref, o_ref, sc_ref,
              x0_ref, y0_ref, x1_ref, y1_ref, s_ref, lab_ref):
    B = o_ref.shape[0]
    sc = sc_ref[...]
    cls = jnp.max(sc, axis=0)
    cio = lax.broadcasted_iota(jnp.int32, sc.shape, 0)
    lbl = jnp.min(jnp.where(sc == cls, cio, sc.shape[0]), axis=0)

    obj = o_ref[...]
    score = cls * obj
    selm = (obj >= 0.5) & (score >= _SCORE_TH)

    xc = coords_ref[0]
    yc = coords_ref[1]
    w = coords_ref[2]
    h = coords_ref[3]
    x0p = jnp.clip(xc - w * 0.5, 0.0, 1.0)
    y0p = jnp.clip(yc - h * 0.5, 0.0, 1.0)
    x1p = jnp.clip(xc + w * 0.5, 0.0, 1.0)
    y1p = jnp.clip(yc + h * 0.5, 0.0, 1.0)
    ap = jnp.maximum(x1p - x0p, 0.0) * jnp.maximum(y1p - y0p, 0.0)

    idxp = (lax.broadcasted_iota(jnp.int32, (1, _R, _C), 1) * _C
            + lax.broadcasted_iota(jnp.int32, (1, _R, _C), 2))
    ms0 = jnp.where(selm, score, -1.0)
    keep0 = jnp.zeros((B, _R, _C), jnp.float32)

    def cond(c):
        ms, _ = c
        return jnp.max(ms) >= 0.0

    def body(c):
        ms, keep = c
        m = jnp.max(ms, axis=(1, 2), keepdims=True)
        active = m >= 0.0
        pidx = jnp.min(jnp.where(ms == m, idxp, _N),
                       axis=(1, 2), keepdims=True)
        onehot = idxp == pidx
        neg = jnp.float32(-1.0)
        px0 = jnp.max(jnp.where(onehot, x0p, neg), axis=(1, 2), keepdims=True)
        py0 = jnp.max(jnp.where(onehot, y0p, neg), axis=(1, 2), keepdims=True)
        px1 = jnp.max(jnp.where(onehot, x1p, neg), axis=(1, 2), keepdims=True)
        py1 = jnp.max(jnp.where(onehot, y1p, neg), axis=(1, 2), keepdims=True)
        pa = jnp.max(jnp.where(onehot, ap, neg), axis=(1, 2), keepdims=True)
        iw = jnp.maximum(jnp.minimum(px1, x1p) - jnp.maximum(px0, x0p), 0.0)
        ih = jnp.maximum(jnp.minimum(py1, y1p) - jnp.maximum(py0, y0p), 0.0)
        inter = iw * ih
        supp = inter * (1.0 + _NMS_TH) > _NMS_TH * (pa + ap)
        keep = jnp.where(onehot & active, 1.0, keep)
        ms = jnp.where(supp | onehot, -1.0, ms)
        return ms, keep

    _, kf = lax.while_loop(cond, body, (ms0, keep0))

    keep = kf > 0.0
    x0_ref[...] = x0p * kf
    y0_ref[...] = y0p * kf
    x1_ref[...] = x1p * kf
    y1_ref[...] = y1p * kf
    s_ref[...] = score * kf
    lab_ref[...] = jnp.where(keep, lbl, 0)


def kernel(b_coords, b_o, b_scores):
    B, N, _ = b_coords.shape
    coords_t = b_coords.transpose(2, 0, 1).reshape(4, B, _R, _C)
    o3 = b_o.reshape(B, _R, _C)
    scores_t = b_scores.transpose(2, 0, 1).reshape(80, B, _R, _C)

    p = jax.ShapeDtypeStruct((B, _R, _C), jnp.float32)
    x0, y0, x1, y1, s, lab = pl.pallas_call(
        _nms_body,
        out_shape=[p, p, p, p, p, jax.ShapeDtypeStruct((B, _R, _C), jnp.int32)],
    )(coords_t, o3, scores_t)

    out = jnp.stack([a.reshape(B, N) for a in (x0, y0, x1, y1, s)], axis=-1)
    labels = lab.reshape(B, N)
    return out, labels
